# baseline (speedup 1.0000x reference)
import numpy as np
import jax
import jax.numpy as jnp
from jax.experimental import pallas as pl

_T = 12


def _fused_body(a_ref, q3_ref, cw_ref, cb_ref, o_ref):
    A = a_ref[...]
    n = A.shape[0]
    rown = jax.lax.broadcasted_iota(jnp.int32, (n, n), 0)
    coln = jax.lax.broadcasted_iota(jnp.int32, (n, n), 1)
    A0 = jnp.where(rown == coln, 0.0, A)
    deg = jnp.sum(A0, axis=1, keepdims=True)
    d = jnp.where(deg > 0, jax.lax.rsqrt(deg), 0.0)
    Ab = A0.astype(jnp.bfloat16)

    q = q3_ref[0]
    u1 = q3_ref[1]
    u2 = q3_ref[2]

    v = -d * jnp.dot(Ab, (d * u2).astype(jnp.bfloat16),
                     preferred_element_type=jnp.float32)
    p = u1 + 2.0 * v
    w = -d * jnp.dot(Ab, (d * p).astype(jnp.bfloat16),
                     preferred_element_type=jnp.float32)
    out = jnp.maximum(q + w, 0.0)

    z = jnp.zeros((n, 1), dtype=out.dtype)
    left = jnp.concatenate([z, out[:, :-1]], axis=1)
    right = jnp.concatenate([out[:, 1:], z], axis=1)
    colt = jax.lax.broadcasted_iota(jnp.int32, (1, out.shape[1]), 1) % _T
    mfirst = (colt != 0).astype(out.dtype)
    mlast = (colt != _T - 1).astype(out.dtype)
    cw = cw_ref[...]
    y = (cw[:, 1:2] * out
         + cw[:, 0:1] * (mfirst * left)
         + cw[:, 2:3] * (mlast * right)
         + cb_ref[0, 0])
    o_ref[...] = jnp.maximum(y, 0.0)


def kernel(X, A, W, b_gcn, conv_w, conv_b):
    B, N, _, T1 = X.shape
    K, _, T2 = W.shape
    x3 = X.reshape(B, N, T1)
    Wq = jnp.stack([W[0] - W[2], W[1], W[2]])
    bias3 = jnp.concatenate(
        [b_gcn.reshape(1, T2), jnp.zeros((2, T2), b_gcn.dtype)])
    q3 = (jnp.einsum('bnt,ktu->knbu', x3, Wq)
          + bias3[:, None, None, :]).reshape(K, N, B * T2)

    y = pl.pallas_call(
        _fused_body,
        out_shape=jax.ShapeDtypeStruct((N, B * T2), X.dtype),
    )(A, q3, conv_w.reshape(1, K), conv_b.reshape(1, 1))
    return y.reshape(N, B, T2).transpose(1, 0, 2).reshape(B, N, 1, T2)
